# Initial kernel scaffold; baseline (speedup 1.0000x reference)
import jax
import jax.numpy as jnp
from jax.experimental import pallas as pl


def kernel(features, states, Uw, Vw, Ww, keys, prelu_a):
    raise NotImplementedError("write your pallas kernel here")


# speedup vs baseline: 2.0950x; 2.0950x over previous
import jax
import jax.numpy as jnp
from jax.experimental import pallas as pl
from jax.experimental.pallas import tpu as pltpu

_BT = 256


def _memory_cell_body(enc_ref, h_ref, keys_ref, Uw_ref, Vw_ref, Ww_ref, a_ref,
                      out_ref):
    g = pl.program_id(0)
    nb = h_ref.shape[0]
    dn = (((1,), (1,)), ((), ()))

    enc = enc_ref[...]
    h = h_ref[...]
    hk = h + keys_ref[...]

    gateT = jax.nn.sigmoid(
        jax.lax.dot_general(hk, enc, dn, preferred_element_type=jnp.float32))
    uhvk = (jax.lax.dot_general(h, Uw_ref[...], dn,
                                preferred_element_type=jnp.float32)
            + jax.lax.dot_general(keys_ref[...], Vw_ref[...], dn,
                                  preferred_element_type=jnp.float32))
    enc_t = enc_ref[pl.ds(g * _BT, _BT), :]
    ew = jax.lax.dot_general(enc_t, Ww_ref[...], dn,
                             preferred_element_type=jnp.float32)

    a = a_ref[...]
    one = jnp.float32(1.0)
    for i in range(nb):
        pre = ew + uhvk[i, :][None, :]
        cand = jnp.where(pre >= 0, pre, a * pre)
        val = h[i, :][None, :] + gateT[i, :][None, :] * cand
        out_ref[i, :, :] = jnp.where(val >= 0, one, -one)


def kernel(features, states, Uw, Vw, Ww, keys, prelu_a):
    B, T, H = features.shape
    NB = keys.shape[0]
    enc = features[:, 0, :]
    h = states.reshape(NB, H)
    a2 = prelu_a.reshape(1, H)

    out = pl.pallas_call(
        _memory_cell_body,
        out_shape=jax.ShapeDtypeStruct((NB, B, H), jnp.float32),
        grid=(B // _BT,),
        in_specs=[
            pl.BlockSpec((B, H), lambda g: (0, 0)),
            pl.BlockSpec((NB, H), lambda g: (0, 0)),
            pl.BlockSpec((NB, H), lambda g: (0, 0)),
            pl.BlockSpec((H, H), lambda g: (0, 0)),
            pl.BlockSpec((H, H), lambda g: (0, 0)),
            pl.BlockSpec((H, H), lambda g: (0, 0)),
            pl.BlockSpec((1, H), lambda g: (0, 0)),
        ],
        out_specs=pl.BlockSpec((NB, _BT, H), lambda g: (0, g, 0)),
        compiler_params=pltpu.CompilerParams(
            dimension_semantics=("parallel",),
        ),
        name="memory_cell",
    )(enc, h, keys, Uw, Vw, Ww, a2)
    return out.reshape(NB * B, H)
